# baseline (speedup 1.0000x reference)
import functools
import jax
import jax.numpy as jnp
from jax import lax
from jax.experimental import pallas as pl
from jax.experimental.pallas import tpu as pltpu
from jax.experimental.pallas import tpu_sc as plsc

B = 4096
L = 200
E = 64
LP1 = L + 1

NC = 2
NS = 16
NW = NC * NS
BPW = B // NW
G = 2
R = BPW // G
HALF = L // 2
LANES = 16


def _body(x_hbm, subject_hbm, table_hbm, subject_table_hbm, out_hbm,
          idx_v, gbuf, sidx_v, pidx_v, srow_v, ssem, gsem0, gsem1,
          wsem0, wsem1):
    wid = lax.axis_index("s") * NC + lax.axis_index("c")
    base = wid * BPW

    pltpu.sync_copy(subject_hbm.at[pl.ds(base, BPW)], sidx_v)
    for k in range(BPW // LANES):
        lane = lax.iota(jnp.int32, LANES)
        pidx_v[pl.ds(k * LANES, LANES)] = (base + k * LANES + lane) * LP1
    pltpu.async_copy(subject_table_hbm.at[sidx_v], srow_v, ssem).wait()
    sub_scatter = pltpu.async_copy(srow_v, out_hbm.at[pidx_v], ssem)

    gsems = (gsem0, gsem1)
    wsems = (wsem0, wsem1)

    def start_round(r, q):
        pltpu.sync_copy(x_hbm.at[pl.ds(base + r * G, G)], idx_v.at[q])
        for g in range(G):
            for h in range(2):
                pltpu.async_copy(
                    table_hbm.at[idx_v.at[q, g, h]],
                    gbuf.at[q, g, pl.ds(h * HALF, HALF)],
                    gsems[q])

    def wait_gathers(q):
        for g in range(G):
            for h in range(2):
                pltpu.make_async_copy(
                    table_hbm.at[idx_v.at[q, g, h]],
                    gbuf.at[q, g, pl.ds(h * HALF, HALF)],
                    gsems[q]).wait()

    def start_writes(r, q):
        for g in range(G):
            row0 = (base + r * G + g) * LP1 + 1
            pltpu.async_copy(gbuf.at[q, g], out_hbm.at[pl.ds(row0, L)],
                             wsems[q])

    def wait_writes(q):
        for g in range(G):
            pltpu.make_async_copy(gbuf.at[q, g],
                                  out_hbm.at[pl.ds(1, L)],
                                  wsems[q]).wait()

    start_round(0, 0)

    @pl.loop(0, R, step=2)
    def _rounds(r):
        for q in (0, 1):
            rr = r + q
            nxt = 1 - q

            @pl.when(rr + 1 < R)
            def _():
                @pl.when(rr >= 1)
                def _():
                    wait_writes(nxt)
                start_round(rr + 1, nxt)

            wait_gathers(q)
            start_writes(rr, q)

    wait_writes(0)
    wait_writes(1)
    sub_scatter.wait()


@jax.jit
def _run(x3, subject, table, subject_table):
    kern = functools.partial(
        pl.kernel,
        out_type=jax.ShapeDtypeStruct((B * LP1, E), jnp.float32),
        mesh=plsc.VectorSubcoreMesh(
            core_axis_name="c", subcore_axis_name="s",
            num_cores=NC, num_subcores=NS),
        scratch_types=[
            pltpu.VMEM((2, G, 2, HALF), jnp.int32),
            pltpu.VMEM((2, G, L, E), jnp.float32),
            pltpu.VMEM((BPW,), jnp.int32),
            pltpu.VMEM((BPW,), jnp.int32),
            pltpu.VMEM((BPW, E), jnp.float32),
            pltpu.SemaphoreType.DMA,
            pltpu.SemaphoreType.DMA,
            pltpu.SemaphoreType.DMA,
            pltpu.SemaphoreType.DMA,
            pltpu.SemaphoreType.DMA,
        ],
        compiler_params=pltpu.CompilerParams(use_tc_tiling_on_sc=False),
    )(_body)
    return kern(x3, subject, table, subject_table)


def kernel(x, subject, table, subject_table):
    x3 = x.reshape(B, 2, HALF)
    out = _run(x3, subject, table, subject_table)
    return out.reshape(B, LP1, E)


# speedup vs baseline: 1.0013x; 1.0013x over previous
import functools
import jax
import jax.numpy as jnp
from jax import lax
from jax.experimental import pallas as pl
from jax.experimental.pallas import tpu as pltpu
from jax.experimental.pallas import tpu_sc as plsc

B = 4096
L = 200
E = 64
LP1 = L + 1

NC = 2
NS = 16
NW = NC * NS
BPW = B // NW
G = 2
R = BPW // G
S1 = 104
S2 = L - S1


def _body(x_hbm, subject_hbm, table_hbm, subject_table_hbm, out_hbm,
          idx_v, gbuf, sidx_v, srow_v, ssem, gsem0, gsem1, wsem0, wsem1):
    wid = lax.axis_index("s") * NC + lax.axis_index("c")
    base = wid * BPW

    pltpu.sync_copy(subject_hbm.at[pl.ds(base, BPW)], sidx_v)
    pltpu.async_copy(subject_table_hbm.at[sidx_v], srow_v, ssem).wait()

    gsems = (gsem0, gsem1)
    wsems = (wsem0, wsem1)

    def start_round(r, q):
        pltpu.sync_copy(x_hbm.at[pl.ds(base + r * G, G)], idx_v.at[q])
        for g in range(G):
            for off, n in ((0, S1), (S1, S2)):
                pltpu.async_copy(
                    table_hbm.at[idx_v.at[q, g, pl.ds(off, n)]],
                    gbuf.at[q, g, pl.ds(1 + off, n)],
                    gsems[q])

    def wait_gathers(q):
        for g in range(G):
            for off, n in ((0, S1), (S1, S2)):
                pltpu.make_async_copy(
                    table_hbm.at[idx_v.at[q, g, pl.ds(off, n)]],
                    gbuf.at[q, g, pl.ds(1 + off, n)],
                    gsems[q]).wait()

    def start_writes(r, q):
        for g in range(G):
            for k in range(E // 16):
                gbuf[q, g, 0, pl.ds(k * 16, 16)] = (
                    srow_v[r * G + g, pl.ds(k * 16, 16)])
            pltpu.async_copy(gbuf.at[q, g], out_hbm.at[base + r * G + g],
                             wsems[q])

    def wait_writes(q):
        for g in range(G):
            pltpu.make_async_copy(gbuf.at[q, g], out_hbm.at[0],
                                  wsems[q]).wait()

    start_round(0, 0)

    @pl.loop(0, R, step=2)
    def _rounds(r):
        for q in (0, 1):
            rr = r + q
            nxt = 1 - q

            @pl.when(rr + 1 < R)
            def _():
                @pl.when(rr >= 1)
                def _():
                    wait_writes(nxt)
                start_round(rr + 1, nxt)

            wait_gathers(q)
            start_writes(rr, q)

    wait_writes(0)
    wait_writes(1)


@jax.jit
def _run(x, subject, table, subject_table):
    kern = functools.partial(
        pl.kernel,
        out_type=jax.ShapeDtypeStruct((B, LP1, E), jnp.float32),
        mesh=plsc.VectorSubcoreMesh(
            core_axis_name="c", subcore_axis_name="s",
            num_cores=NC, num_subcores=NS),
        scratch_types=[
            pltpu.VMEM((2, G, L), jnp.int32),
            pltpu.VMEM((2, G, LP1, E), jnp.float32),
            pltpu.VMEM((BPW,), jnp.int32),
            pltpu.VMEM((BPW, E), jnp.float32),
            pltpu.SemaphoreType.DMA,
            pltpu.SemaphoreType.DMA,
            pltpu.SemaphoreType.DMA,
            pltpu.SemaphoreType.DMA,
            pltpu.SemaphoreType.DMA,
        ],
        compiler_params=pltpu.CompilerParams(use_tc_tiling_on_sc=False),
    )(_body)
    return kern(x, subject, table, subject_table)


def kernel(x, subject, table, subject_table):
    return _run(x, subject, table, subject_table)
